# baseline (speedup 1.0000x reference)
import functools

import jax
import jax.numpy as jnp
from jax import lax
from jax.experimental import pallas as pl
from jax.experimental.pallas import tpu as pltpu
from jax.experimental.pallas import tpu_sc as plsc

N_NODES = 10000
N_EDGES = 320000
D = 128
HEADS = 4
C = 32
AW = 16

NUM_CORES = 2
NUM_TILES = 16
NW = NUM_CORES * NUM_TILES
BLK = 80
E_PER_W = 10240
E_REAL_W = N_EDGES // NW
N_BLKS = E_PER_W // BLK
E_PAD = E_PER_W * NW
N_PAD = 10240
ROWS_PER_TILE = N_PAD // NUM_TILES


def _prep_body(x_ref, w_ref, o_ref):
    o_ref[...] = lax.dot_general(
        x_ref[...], w_ref[...], (((1,), (0,)), ((), ())),
        preferred_element_type=jnp.float32, precision=lax.Precision.HIGHEST)


_tc_prep = pl.pallas_call(
    _prep_body,
    out_shape=jax.ShapeDtypeStruct((N_NODES, AW), jnp.float32),
)


def _comb_body(p_ref, o_ref):
    o_ref[...] = p_ref[0, :N_NODES] + p_ref[1, :N_NODES]


_tc_combine = pl.pallas_call(
    _comb_body,
    out_shape=jax.ShapeDtypeStruct((N_NODES, D), jnp.float32),
)


def _sc_body(a_hbm, sde_hbm, x_hbm, zeros_hbm, out_hbm,
             idxew_v, ai_v, aj_v, xg_v, xs_v, betat_v, shared_out,
             sem_i0, sem_i1, sem_i2, sem_i3,
             sem_a0, sem_a1, sem_b0, sem_b1, sem_x0, sem_x1,
             sem_s0, sem_s1):
    c = lax.axis_index("c")
    s = lax.axis_index("s")
    tid = c * NUM_TILES + s
    edge0 = tid * E_PER_W

    sem_i = [sem_i0, sem_i1, sem_i2, sem_i3]
    sem_a = [sem_a0, sem_a1]
    sem_b = [sem_b0, sem_b1]
    sem_x = [sem_x0, sem_x1]
    sem_s = [sem_s0, sem_s1]

    pltpu.sync_copy(zeros_hbm, shared_out.at[pl.ds(s * ROWS_PER_TILE, ROWS_PER_TILE)])
    plsc.subcore_barrier()

    def idx_copy(i, q):
        return pltpu.make_async_copy(
            sde_hbm.at[:, pl.ds(edge0 + i * BLK, BLK)], idxew_v.at[q], sem_i[q])

    def gather_copies(p, q):
        return (
            pltpu.make_async_copy(a_hbm.at[idxew_v.at[q, 0]], ai_v.at[p], sem_a[p]),
            pltpu.make_async_copy(a_hbm.at[idxew_v.at[q, 1]], aj_v.at[p], sem_b[p]),
            pltpu.make_async_copy(x_hbm.at[idxew_v.at[q, 1]], xg_v.at[p], sem_x[p]),
        )

    def scatter_copy(p, q):
        return pltpu.make_async_copy(
            xs_v.at[p], shared_out.at[idxew_v.at[q, 0]], sem_s[p])

    def compute(p, q):
        p16 = jnp.full((16,), p, jnp.int32)
        for g in range(BLK // 16):
            sl = pl.ds(g * 16, 16)
            rows = jnp.full((16,), g * 16, jnp.int32) + lax.iota(jnp.int32, 16)
            w16 = plsc.bitcast(idxew_v[q, 2, sl], jnp.float32)
            for h in range(HEADS):
                a1 = plsc.load_gather(
                    ai_v, [p16, rows, jnp.full((16,), h, jnp.int32)])
                a2 = plsc.load_gather(
                    aj_v, [p16, rows, jnp.full((16,), HEADS + h, jnp.int32)])
                beta = w16 / (1.0 + jnp.exp(-(a1 + a2)))
                betat_v[pl.ds(h * BLK + g * 16, 16)] = beta

        def edge(e, carry2):
            e_idx = jnp.full((16,), e, jnp.int32)
            for h in range(HEADS):
                b = plsc.load_gather(betat_v, [e_idx + (h * BLK)])
                for k in range(C // 16):
                    fsl = pl.ds(h * C + k * 16, 16)
                    xs_v[p, e, fsl] = xg_v[p, e, fsl] * b
            return carry2

        lax.fori_loop(0, BLK, edge, 0)

    idx_copy(0, 0).start()
    idx_copy(1, 1).start()
    idx_copy(0, 0).wait()
    for cpy in gather_copies(0, 0):
        cpy.start()

    def quad(j, carry):
        for q in range(4):
            i = j * 4 + q
            pslot = q % 2

            if q <= 1:
                @pl.when(j > 0)
                def _():
                    scatter_copy(pslot, (q + 2) % 4).wait()
            else:
                scatter_copy(pslot, (q + 2) % 4).wait()

            def launch_next():
                idx_copy(i + 1, (q + 1) % 4).wait()
                for cpy in gather_copies(pslot ^ 1, (q + 1) % 4):
                    cpy.start()

            if q == 3:
                @pl.when(j < (N_BLKS // 4) - 1)
                def _():
                    launch_next()
            else:
                launch_next()

            def prefetch_idx():
                idx_copy(i + 2, (q + 2) % 4).start()

            if q >= 2:
                @pl.when(j < (N_BLKS // 4) - 1)
                def _():
                    prefetch_idx()
            else:
                prefetch_idx()

            for cpy in gather_copies(pslot, q):
                cpy.wait()
            compute(pslot, q)
            scatter_copy(pslot, q).start(add=True)
        return carry

    lax.fori_loop(0, N_BLKS // 4, quad, 0)
    scatter_copy(0, (N_BLKS - 2) % 4).wait()
    scatter_copy(1, (N_BLKS - 1) % 4).wait()

    plsc.subcore_barrier()

    rsl = pl.ds(s * ROWS_PER_TILE, ROWS_PER_TILE)
    pltpu.sync_copy(shared_out.at[rsl], out_hbm.at[c, rsl])


_sc_main = functools.partial(
    pl.kernel,
    out_type=jax.ShapeDtypeStruct((NUM_CORES, N_PAD, D), jnp.float32),
    mesh=plsc.VectorSubcoreMesh(core_axis_name="c", subcore_axis_name="s"),
    compiler_params=pltpu.CompilerParams(
        needs_layout_passes=False, use_tc_tiling_on_sc=False),
    scratch_types=[
        pltpu.VMEM((4, 3, BLK), jnp.int32),
        pltpu.VMEM((2, BLK, AW), jnp.float32),
        pltpu.VMEM((2, BLK, AW), jnp.float32),
        pltpu.VMEM((2, BLK, D), jnp.float32),
        pltpu.VMEM((2, BLK, D), jnp.float32),
        pltpu.VMEM((HEADS * BLK,), jnp.float32),
        pltpu.VMEM_SHARED((N_PAD, D), jnp.float32),
    ] + [pltpu.SemaphoreType.DMA] * 12,
)(_sc_body)


def kernel(x_tangent0, edge_index, edge_weight, W):
    src = edge_index[0].astype(jnp.int32)
    dst = edge_index[1].astype(jnp.int32)
    ew_bits = lax.bitcast_convert_type(edge_weight, jnp.int32)
    pad_cols = E_PER_W - E_REAL_W
    sde = jnp.stack([
        jnp.pad(src.reshape(NW, E_REAL_W), ((0, 0), (0, pad_cols))).reshape(-1),
        jnp.pad(dst.reshape(NW, E_REAL_W), ((0, 0), (0, pad_cols))).reshape(-1),
        jnp.pad(ew_bits.reshape(NW, E_REAL_W), ((0, 0), (0, pad_cols))).reshape(-1),
    ])
    w1 = W[0, :C]
    w2 = W[0, C:]
    eye = jnp.eye(HEADS, dtype=jnp.float32)
    wm = jnp.concatenate(
        [jnp.kron(eye, w1[:, None]), jnp.kron(eye, w2[:, None]),
         jnp.zeros((D, AW - 2 * HEADS), jnp.float32)], axis=1)
    a = _tc_prep(x_tangent0, wm)
    zeros = jnp.zeros((ROWS_PER_TILE, D), jnp.float32)
    partials = _sc_main(a, sde, x_tangent0, zeros)
    return _tc_combine(partials)


# speedup vs baseline: 1.2962x; 1.2962x over previous
import functools

import jax
import jax.numpy as jnp
from jax import lax
from jax.experimental import pallas as pl
from jax.experimental.pallas import tpu as pltpu
from jax.experimental.pallas import tpu_sc as plsc

N_NODES = 10000
N_EDGES = 320000
D = 128
HEADS = 4
C = 32

NUM_CORES = 2
NUM_TILES = 16
NW = NUM_CORES * NUM_TILES
BLK = 32
E_PER_W = 10176
E_REAL_W = N_EDGES // NW
N_BLKS = E_PER_W // BLK
E_PAD = E_PER_W * NW
N_PAD = 10240
ROWS_PER_TILE = N_PAD // NUM_TILES

_HI_MASK = -65536


def _prep_body(x_ref, w1_ref, w2_ref, o_ref):
    a1 = lax.dot_general(
        x_ref[...], w1_ref[...], (((1,), (0,)), ((), ())),
        preferred_element_type=jnp.float32, precision=lax.Precision.HIGHEST)
    a2 = lax.dot_general(
        x_ref[...], w2_ref[...], (((1,), (0,)), ((), ())),
        preferred_element_type=jnp.float32, precision=lax.Precision.HIGHEST)
    b1 = lax.bitcast_convert_type(a1, jnp.int32)
    b2 = lax.bitcast_convert_type(a2, jnp.int32)
    r1 = (b1 + 0x8000) & _HI_MASK
    r2 = lax.shift_right_logical(b2 + 0x8000, 16)
    o_ref[...] = lax.transpose(r1 | r2, (1, 0))


_tc_prep = pl.pallas_call(
    _prep_body,
    out_shape=jax.ShapeDtypeStruct((HEADS, N_NODES), jnp.int32),
)


def _comb_body(p_ref, o_ref):
    o_ref[...] = p_ref[0, :N_NODES] + p_ref[1, :N_NODES]


_tc_combine = pl.pallas_call(
    _comb_body,
    out_shape=jax.ShapeDtypeStruct((N_NODES, D), jnp.float32),
)


def _sc_body(tab_hbm, sde_hbm, x_hbm, zeros_hbm, out_hbm,
             tab_v, idxew_v, xj_v, betat_v, shared_out,
             sem_i0, sem_i1, sem_i2, sem_x0, sem_x1):
    c = lax.axis_index("c")
    s = lax.axis_index("s")
    tid = c * NUM_TILES + s
    edge0 = tid * E_PER_W

    sem_i = [sem_i0, sem_i1, sem_i2]
    sem_x = [sem_x0, sem_x1]

    pltpu.sync_copy(zeros_hbm, shared_out.at[pl.ds(s * ROWS_PER_TILE, ROWS_PER_TILE)])
    pltpu.sync_copy(tab_hbm, tab_v)
    plsc.subcore_barrier()

    def idx_copy(i, q):
        return pltpu.make_async_copy(
            sde_hbm.at[:, pl.ds(edge0 + i * BLK, BLK)], idxew_v.at[q], sem_i[q])

    def gather_copy(p, q):
        return pltpu.make_async_copy(
            x_hbm.at[idxew_v.at[q, 1]], xj_v.at[p], sem_x[p])

    def block(i, p, q, j, u):
        def launch_next():
            idx_copy(i + 1, (q + 1) % 3).wait()
            gather_copy(p ^ 1, (q + 1) % 3).start()

        if u == 5:
            @pl.when(j < (N_BLKS // 6) - 1)
            def _():
                launch_next()
        else:
            launch_next()

        def prefetch_idx():
            idx_copy(i + 2, (q + 2) % 3).start()

        if u >= 4:
            @pl.when(j < (N_BLKS // 6) - 1)
            def _():
                prefetch_idx()
        else:
            prefetch_idx()

        for g in range(BLK // 16):
            sl = pl.ds(g * 16, 16)
            s16 = idxew_v[q, 0, sl]
            d16 = idxew_v[q, 1, sl]
            w16 = plsc.bitcast(idxew_v[q, 2, sl], jnp.float32)
            for h in range(HEADS):
                ws = plsc.load_gather(tab_v, [s16 + (h * N_NODES)])
                wd = plsc.load_gather(tab_v, [d16 + (h * N_NODES)])
                a1 = plsc.bitcast(ws & _HI_MASK, jnp.float32)
                a2 = plsc.bitcast(lax.shift_left(wd, 16), jnp.float32)
                beta = w16 / (1.0 + jnp.exp(-(a1 + a2)))
                betat_v[pl.ds(h * BLK + g * 16, 16)] = beta

        gather_copy(p, q).wait()

        def edge(e, carry2):
            e_idx = jnp.full((16,), e, jnp.int32)
            for h in range(HEADS):
                b = plsc.load_gather(betat_v, [e_idx + (h * BLK)])
                for k in range(C // 16):
                    fsl = pl.ds(h * C + k * 16, 16)
                    xj_v[p, e, fsl] = xj_v[p, e, fsl] * b
            return carry2

        lax.fori_loop(0, BLK, edge, 0)

        pltpu.sync_copy(xj_v.at[p], shared_out.at[idxew_v.at[q, 0]], add=True)

    idx_copy(0, 0).start()
    idx_copy(1, 1).start()
    idx_copy(0, 0).wait()
    gather_copy(0, 0).start()

    def six(j, carry):
        for u in range(6):
            i = j * 6 + u
            block(i, u % 2, u % 3, j, u)
        return carry

    lax.fori_loop(0, N_BLKS // 6, six, 0)
    plsc.subcore_barrier()

    rsl = pl.ds(s * ROWS_PER_TILE, ROWS_PER_TILE)
    pltpu.sync_copy(shared_out.at[rsl], out_hbm.at[c, rsl])


_sc_main = functools.partial(
    pl.kernel,
    out_type=jax.ShapeDtypeStruct((NUM_CORES, N_PAD, D), jnp.float32),
    mesh=plsc.VectorSubcoreMesh(core_axis_name="c", subcore_axis_name="s"),
    compiler_params=pltpu.CompilerParams(
        needs_layout_passes=False, use_tc_tiling_on_sc=False),
    scratch_types=[
        pltpu.VMEM((HEADS * N_NODES,), jnp.int32),
        pltpu.VMEM((3, 3, BLK), jnp.int32),
        pltpu.VMEM((2, BLK, D), jnp.float32),
        pltpu.VMEM((HEADS * BLK,), jnp.float32),
        pltpu.VMEM_SHARED((N_PAD, D), jnp.float32),
        pltpu.SemaphoreType.DMA,
        pltpu.SemaphoreType.DMA,
        pltpu.SemaphoreType.DMA,
        pltpu.SemaphoreType.DMA,
        pltpu.SemaphoreType.DMA,
    ],
)(_sc_body)


def kernel(x_tangent0, edge_index, edge_weight, W):
    src = edge_index[0].astype(jnp.int32)
    dst = edge_index[1].astype(jnp.int32)
    ew_bits = lax.bitcast_convert_type(edge_weight, jnp.int32)
    pad_cols = E_PER_W - E_REAL_W
    sde = jnp.stack([
        jnp.pad(src.reshape(NW, E_REAL_W), ((0, 0), (0, pad_cols))).reshape(-1),
        jnp.pad(dst.reshape(NW, E_REAL_W), ((0, 0), (0, pad_cols))).reshape(-1),
        jnp.pad(ew_bits.reshape(NW, E_REAL_W), ((0, 0), (0, pad_cols))).reshape(-1),
    ])
    w1 = W[0, :C]
    w2 = W[0, C:]
    eye = jnp.eye(HEADS, dtype=jnp.float32)
    wm1 = jnp.kron(eye, w1[:, None])
    wm2 = jnp.kron(eye, w2[:, None])
    tab = _tc_prep(x_tangent0, wm1, wm2).reshape(-1)
    zeros = jnp.zeros((ROWS_PER_TILE, D), jnp.float32)
    partials = _sc_main(tab, sde, x_tangent0, zeros)
    return _tc_combine(partials)


# speedup vs baseline: 1.3365x; 1.0311x over previous
import functools

import jax
import jax.numpy as jnp
from jax import lax
from jax.experimental import pallas as pl
from jax.experimental.pallas import tpu as pltpu
from jax.experimental.pallas import tpu_sc as plsc

N_NODES = 10000
N_EDGES = 320000
D = 128
HEADS = 4
C = 32

NUM_CORES = 2
NUM_TILES = 16
NW = NUM_CORES * NUM_TILES
BLK = 64
E_PER_W = 10048
E_REAL_W = N_EDGES // NW
N_BLKS = E_PER_W // BLK
E_PAD = E_PER_W * NW
N_PAD = 10240
ROWS_PER_TILE = N_PAD // NUM_TILES

_HI_MASK = -65536


def _prep_body(x_ref, w1_ref, w2_ref, o_ref):
    a1 = lax.dot_general(
        x_ref[...], w1_ref[...], (((1,), (0,)), ((), ())),
        preferred_element_type=jnp.float32, precision=lax.Precision.HIGHEST)
    a2 = lax.dot_general(
        x_ref[...], w2_ref[...], (((1,), (0,)), ((), ())),
        preferred_element_type=jnp.float32, precision=lax.Precision.HIGHEST)
    b1 = lax.bitcast_convert_type(a1, jnp.int32)
    b2 = lax.bitcast_convert_type(a2, jnp.int32)
    r1 = (b1 + 0x8000) & _HI_MASK
    r2 = lax.shift_right_logical(b2 + 0x8000, 16)
    o_ref[...] = lax.transpose(r1 | r2, (1, 0))


_tc_prep = pl.pallas_call(
    _prep_body,
    out_shape=jax.ShapeDtypeStruct((HEADS, N_NODES), jnp.int32),
)


def _comb_body(p_ref, o_ref):
    o_ref[...] = p_ref[0, :N_NODES] + p_ref[1, :N_NODES]


_tc_combine = pl.pallas_call(
    _comb_body,
    out_shape=jax.ShapeDtypeStruct((N_NODES, D), jnp.float32),
)


def _sc_body(tab_hbm, sde_hbm, x_hbm, zeros_hbm, out_hbm,
             tab_v, idxew_v, xj_v, betat_v, shared_out, sem_x):
    c = lax.axis_index("c")
    s = lax.axis_index("s")
    tid = c * NUM_TILES + s
    edge0 = tid * E_PER_W

    pltpu.sync_copy(zeros_hbm, shared_out.at[pl.ds(s * ROWS_PER_TILE, ROWS_PER_TILE)])
    pltpu.sync_copy(tab_hbm, tab_v)
    plsc.subcore_barrier()

    def block(i, carry):
        base = edge0 + i * BLK
        pltpu.sync_copy(sde_hbm.at[:, pl.ds(base, BLK)], idxew_v)
        cx = pltpu.async_copy(x_hbm.at[idxew_v.at[1]], xj_v, sem_x)

        for g in range(BLK // 16):
            sl = pl.ds(g * 16, 16)
            s16 = idxew_v[0, sl]
            d16 = idxew_v[1, sl]
            w16 = plsc.bitcast(idxew_v[2, sl], jnp.float32)
            for h in range(HEADS):
                ws = plsc.load_gather(tab_v, [s16 + (h * N_NODES)])
                wd = plsc.load_gather(tab_v, [d16 + (h * N_NODES)])
                a1 = plsc.bitcast(ws & _HI_MASK, jnp.float32)
                a2 = plsc.bitcast(lax.shift_left(wd, 16), jnp.float32)
                beta = w16 / (1.0 + jnp.exp(-(a1 + a2)))
                betat_v[pl.ds(h * BLK + g * 16, 16)] = beta

        cx.wait()

        def edge8(t, carry2):
            e0 = t * 8
            for r in range(8):
                e = e0 + r
                e_idx = jnp.full((16,), r, jnp.int32) + e0
                for h in range(HEADS):
                    b = plsc.load_gather(betat_v, [e_idx + (h * BLK)])
                    for k in range(C // 16):
                        fsl = pl.ds(h * C + k * 16, 16)
                        xj_v[e, fsl] = xj_v[e, fsl] * b
            return carry2

        lax.fori_loop(0, BLK // 8, edge8, 0)

        pltpu.sync_copy(xj_v, shared_out.at[idxew_v.at[0]], add=True)
        return carry

    lax.fori_loop(0, N_BLKS, block, 0)
    plsc.subcore_barrier()

    rsl = pl.ds(s * ROWS_PER_TILE, ROWS_PER_TILE)
    pltpu.sync_copy(shared_out.at[rsl], out_hbm.at[c, rsl])


_sc_main = functools.partial(
    pl.kernel,
    out_type=jax.ShapeDtypeStruct((NUM_CORES, N_PAD, D), jnp.float32),
    mesh=plsc.VectorSubcoreMesh(core_axis_name="c", subcore_axis_name="s"),
    compiler_params=pltpu.CompilerParams(
        needs_layout_passes=False, use_tc_tiling_on_sc=False),
    scratch_types=[
        pltpu.VMEM((HEADS * N_NODES,), jnp.int32),
        pltpu.VMEM((3, BLK), jnp.int32),
        pltpu.VMEM((BLK, D), jnp.float32),
        pltpu.VMEM((HEADS * BLK,), jnp.float32),
        pltpu.VMEM_SHARED((N_PAD, D), jnp.float32),
        pltpu.SemaphoreType.DMA,
    ],
)(_sc_body)


def kernel(x_tangent0, edge_index, edge_weight, W):
    src = edge_index[0].astype(jnp.int32)
    dst = edge_index[1].astype(jnp.int32)
    ew_bits = lax.bitcast_convert_type(edge_weight, jnp.int32)
    pad_cols = E_PER_W - E_REAL_W
    sde = jnp.stack([
        jnp.pad(src.reshape(NW, E_REAL_W), ((0, 0), (0, pad_cols))).reshape(-1),
        jnp.pad(dst.reshape(NW, E_REAL_W), ((0, 0), (0, pad_cols))).reshape(-1),
        jnp.pad(ew_bits.reshape(NW, E_REAL_W), ((0, 0), (0, pad_cols))).reshape(-1),
    ])
    w1 = W[0, :C]
    w2 = W[0, C:]
    eye = jnp.eye(HEADS, dtype=jnp.float32)
    wm1 = jnp.kron(eye, w1[:, None])
    wm2 = jnp.kron(eye, w2[:, None])
    tab = _tc_prep(x_tangent0, wm1, wm2).reshape(-1)
    zeros = jnp.zeros((ROWS_PER_TILE, D), jnp.float32)
    partials = _sc_main(tab, sde, x_tangent0, zeros)
    return _tc_combine(partials)


# speedup vs baseline: 1.7709x; 1.3250x over previous
import functools

import jax
import jax.numpy as jnp
from jax import lax
from jax.experimental import pallas as pl
from jax.experimental.pallas import tpu as pltpu
from jax.experimental.pallas import tpu_sc as plsc

N_NODES = 10000
N_EDGES = 320000
D = 128
HEADS = 4
C = 32

NUM_CORES = 2
NUM_TILES = 16
NW = NUM_CORES * NUM_TILES
BLK = 64
E_PER_W = 10048
E_REAL_W = N_EDGES // NW
N_BLKS = E_PER_W // BLK
E_PAD = E_PER_W * NW
N_PAD = 10240
ROWS_PER_TILE = N_PAD // NUM_TILES

_HI_MASK = -65536


def _prep_body(x_ref, w1_ref, w2_ref, o_ref):
    a1 = lax.dot_general(
        x_ref[...], w1_ref[...], (((1,), (0,)), ((), ())),
        preferred_element_type=jnp.float32, precision=lax.Precision.HIGHEST)
    a2 = lax.dot_general(
        x_ref[...], w2_ref[...], (((1,), (0,)), ((), ())),
        preferred_element_type=jnp.float32, precision=lax.Precision.HIGHEST)
    b1 = lax.bitcast_convert_type(a1, jnp.int32)
    b2 = lax.bitcast_convert_type(a2, jnp.int32)
    r1 = (b1 + 0x8000) & _HI_MASK
    r2 = lax.shift_right_logical(b2 + 0x8000, 16)
    o_ref[...] = lax.transpose(r1 | r2, (1, 0))


_tc_prep = pl.pallas_call(
    _prep_body,
    out_shape=jax.ShapeDtypeStruct((HEADS, N_NODES), jnp.int32),
)


def _comb_body(p_ref, o_ref):
    o_ref[...] = p_ref[0, :N_NODES] + p_ref[1, :N_NODES]


_tc_combine = pl.pallas_call(
    _comb_body,
    out_shape=jax.ShapeDtypeStruct((N_NODES, D), jnp.float32),
)


def _sc_body(tab_hbm, sde_hbm, x_hbm, zeros_hbm, out_hbm,
             tab_v, idxew_v, xj_v, betat_v, shared_out, sem_x):
    c = lax.axis_index("c")
    s = lax.axis_index("s")
    tid = c * NUM_TILES + s
    edge0 = tid * E_PER_W

    pltpu.sync_copy(zeros_hbm, shared_out.at[pl.ds(s * ROWS_PER_TILE, ROWS_PER_TILE)])
    pltpu.sync_copy(tab_hbm, tab_v)
    plsc.subcore_barrier()

    def block(i, carry):
        base = edge0 + i * BLK
        pltpu.sync_copy(sde_hbm.at[:, pl.ds(base, BLK)], idxew_v)
        cx = pltpu.async_copy(x_hbm.at[idxew_v.at[1]], xj_v, sem_x)

        for g in range(BLK // 16):
            sl = pl.ds(g * 16, 16)
            s16 = idxew_v[0, sl]
            d16 = idxew_v[1, sl]
            w16 = plsc.bitcast(idxew_v[2, sl], jnp.float32)
            for h in range(HEADS):
                ws = plsc.load_gather(tab_v, [s16 + (h * N_NODES)])
                wd = plsc.load_gather(tab_v, [d16 + (h * N_NODES)])
                a1 = plsc.bitcast(ws & _HI_MASK, jnp.float32)
                a2 = plsc.bitcast(lax.shift_left(wd, 16), jnp.float32)
                beta = w16 / (1.0 + jnp.exp(-(a1 + a2)))
                betat_v[pl.ds(h * BLK + g * 16, 16)] = beta

        cx.wait()

        def edge8(t, carry2):
            e0 = t * 8
            bs = []
            for r in range(8):
                e_idx = jnp.full((16,), r, jnp.int32) + e0
                bs.append([
                    plsc.load_gather(betat_v, [e_idx + (h * BLK)])
                    for h in range(HEADS)
                ])
            for r in range(8):
                e = e0 + r
                for h in range(HEADS):
                    for k in range(C // 16):
                        fsl = pl.ds(h * C + k * 16, 16)
                        xj_v[e, fsl] = xj_v[e, fsl] * bs[r][h]
            return carry2

        lax.fori_loop(0, BLK // 8, edge8, 0)

        pltpu.sync_copy(xj_v, shared_out.at[idxew_v.at[0]], add=True)
        return carry

    lax.fori_loop(0, N_BLKS, block, 0)
    plsc.subcore_barrier()

    rsl = pl.ds(s * ROWS_PER_TILE, ROWS_PER_TILE)
    pltpu.sync_copy(shared_out.at[rsl], out_hbm.at[c, rsl])


_sc_main = functools.partial(
    pl.kernel,
    out_type=jax.ShapeDtypeStruct((NUM_CORES, N_PAD, D), jnp.float32),
    mesh=plsc.VectorSubcoreMesh(core_axis_name="c", subcore_axis_name="s"),
    compiler_params=pltpu.CompilerParams(
        needs_layout_passes=False, use_tc_tiling_on_sc=False),
    scratch_types=[
        pltpu.VMEM((HEADS * N_NODES,), jnp.int32),
        pltpu.VMEM((3, BLK), jnp.int32),
        pltpu.VMEM((BLK, D), jnp.float32),
        pltpu.VMEM((HEADS * BLK,), jnp.float32),
        pltpu.VMEM_SHARED((N_PAD, D), jnp.float32),
        pltpu.SemaphoreType.DMA,
    ],
)(_sc_body)


def kernel(x_tangent0, edge_index, edge_weight, W):
    src = edge_index[0].astype(jnp.int32)
    dst = edge_index[1].astype(jnp.int32)
    ew_bits = lax.bitcast_convert_type(edge_weight, jnp.int32)
    pad_cols = E_PER_W - E_REAL_W
    sde = jnp.stack([
        jnp.pad(src.reshape(NW, E_REAL_W), ((0, 0), (0, pad_cols))).reshape(-1),
        jnp.pad(dst.reshape(NW, E_REAL_W), ((0, 0), (0, pad_cols))).reshape(-1),
        jnp.pad(ew_bits.reshape(NW, E_REAL_W), ((0, 0), (0, pad_cols))).reshape(-1),
    ])
    w1 = W[0, :C]
    w2 = W[0, C:]
    eye = jnp.eye(HEADS, dtype=jnp.float32)
    wm1 = jnp.kron(eye, w1[:, None])
    wm2 = jnp.kron(eye, w2[:, None])
    tab = _tc_prep(x_tangent0, wm1, wm2).reshape(-1)
    zeros = jnp.zeros((ROWS_PER_TILE, D), jnp.float32)
    partials = _sc_main(tab, sde, x_tangent0, zeros)
    return _tc_combine(partials)
